# baseline (speedup 1.0000x reference)
import jax
import jax.numpy as jnp
from jax.experimental import pallas as pl
from jax.experimental.pallas import tpu as pltpu

_B, _C, _H, _W = 128, 384, 32, 32
_N = 32
_P = _H * _W


def _cluster_body(inf_ref, x_ref, w_ref, o_ref):
    inf = inf_ref[0, 0]
    w = w_ref[...]
    wn = w * jax.lax.rsqrt(
        jnp.maximum(jnp.sum(w * w, axis=1, keepdims=True), 1e-24))
    xb = x_ref[0]
    ssq = jnp.sum(xb * xb, axis=0, keepdims=True)
    inv = jax.lax.rsqrt(jnp.maximum(ssq, 1e-24))
    logits = jnp.dot(wn, xb, preferred_element_type=jnp.float32)
    o_ref[0] = logits * (inv * inf)


def kernel(x, cluster_probe, inference):
    b, c, h, w = x.shape
    n = cluster_probe.shape[0]
    x3 = x.reshape(b, c, h * w)
    inf_arr = jnp.asarray(inference, jnp.float32).reshape(1, 1)
    out = pl.pallas_call(
        _cluster_body,
        grid=(b,),
        in_specs=[
            pl.BlockSpec(memory_space=pltpu.SMEM),
            pl.BlockSpec((1, c, h * w), lambda i: (i, 0, 0)),
            pl.BlockSpec((n, c), lambda i: (0, 0)),
        ],
        out_specs=pl.BlockSpec((1, n, h * w), lambda i: (i, 0, 0)),
        out_shape=jax.ShapeDtypeStruct((b, n, h * w), jnp.float32),
    )(inf_arr, x3, cluster_probe)
    return out.reshape(b, n, h, w)


# speedup vs baseline: 1.2187x; 1.2187x over previous
import jax
import jax.numpy as jnp
from jax.experimental import pallas as pl
from jax.experimental.pallas import tpu as pltpu

_B, _C, _H, _W = 128, 384, 32, 32
_N = 32
_P = _H * _W


_G = 8


def _cluster_body(inf_ref, x_ref, w_ref, o_ref):
    inf = inf_ref[0, 0]
    w = w_ref[...]
    wn = w * jax.lax.rsqrt(
        jnp.maximum(jnp.sum(w * w, axis=1, keepdims=True), 1e-24))
    for g in range(_G):
        xb = x_ref[g]
        ssq = jnp.sum(xb * xb, axis=0, keepdims=True)
        inv = jax.lax.rsqrt(jnp.maximum(ssq, 1e-24))
        logits = jnp.dot(wn, xb, preferred_element_type=jnp.float32)
        o_ref[g] = logits * (inv * inf)


def kernel(x, cluster_probe, inference):
    b, c, h, w = x.shape
    n = cluster_probe.shape[0]
    p = h * w
    x3 = x.reshape(b, c, p)
    inf_arr = jnp.asarray(inference, jnp.float32).reshape(1, 1)
    out = pl.pallas_call(
        _cluster_body,
        grid=(b // _G,),
        in_specs=[
            pl.BlockSpec(memory_space=pltpu.SMEM),
            pl.BlockSpec((_G, c, p), lambda i: (i, 0, 0)),
            pl.BlockSpec((n, c), lambda i: (0, 0)),
        ],
        out_specs=pl.BlockSpec((_G, n, p), lambda i: (i, 0, 0)),
        out_shape=jax.ShapeDtypeStruct((b, n, p), jnp.float32),
    )(inf_arr, x3, cluster_probe)
    return out.reshape(b, n, h, w)


# speedup vs baseline: 2.4422x; 2.0040x over previous
import jax
import jax.numpy as jnp
from jax.experimental import pallas as pl
from jax.experimental.pallas import tpu as pltpu

_MT = 4096


def _cluster_body(inf_ref, x_ref, w_ref, o_ref):
    inf = inf_ref[0, 0]
    wv = w_ref[...]
    wn = wv * jax.lax.rsqrt(
        jnp.maximum(jnp.sum(wv * wv, axis=1, keepdims=True), 1e-24))
    xv = x_ref[...]
    ssq = jnp.sum(xv * xv, axis=1, keepdims=True)
    inv = jax.lax.rsqrt(jnp.maximum(ssq, 1e-24))
    logits = jnp.dot(xv.astype(jnp.bfloat16),
                     wn.T.astype(jnp.bfloat16),
                     preferred_element_type=jnp.float32)
    o_ref[...] = logits * (inv * inf)


def kernel(x, cluster_probe, inference):
    b, c, h, w = x.shape
    n = cluster_probe.shape[0]
    m = b * h * w
    xm = x.transpose(0, 2, 3, 1).reshape(m, c)
    inf_arr = jnp.asarray(inference, jnp.float32).reshape(1, 1)
    out = pl.pallas_call(
        _cluster_body,
        grid=(m // _MT,),
        in_specs=[
            pl.BlockSpec(memory_space=pltpu.SMEM),
            pl.BlockSpec((_MT, c), lambda i: (i, 0)),
            pl.BlockSpec((n, c), lambda i: (0, 0)),
        ],
        out_specs=pl.BlockSpec((_MT, n), lambda i: (i, 0)),
        out_shape=jax.ShapeDtypeStruct((m, n), jnp.float32),
    )(inf_arr, xm, cluster_probe)
    return out.reshape(b, h, w, n).transpose(0, 3, 1, 2)


# speedup vs baseline: 3.3203x; 1.3596x over previous
import jax
import jax.numpy as jnp
from jax.experimental import pallas as pl
from jax.experimental.pallas import tpu as pltpu

_HT = 1


def _cluster_body(inf_ref, x_ref, w_ref, o_ref):
    inf = inf_ref[0, 0]
    wv = w_ref[...]
    wn = wv * jax.lax.rsqrt(
        jnp.maximum(jnp.sum(wv * wv, axis=1, keepdims=True), 1e-24))
    b, ht, w, c = x_ref.shape
    xv = x_ref[...].reshape(b * ht * w, c)
    ssq = jnp.sum(xv * xv, axis=1, keepdims=True)
    inv = jax.lax.rsqrt(jnp.maximum(ssq, 1e-24))
    logits = jnp.dot(xv.astype(jnp.bfloat16),
                     wn.T.astype(jnp.bfloat16),
                     preferred_element_type=jnp.float32)
    scaled = logits * (inv * inf)
    n = scaled.shape[1]
    cube = scaled.reshape(b, ht * w, n)
    o_ref[...] = jnp.transpose(cube, (2, 1, 0))


def kernel(x, cluster_probe, inference):
    b, c, h, w = x.shape
    n = cluster_probe.shape[0]
    xt = x.transpose(0, 2, 3, 1)
    inf_arr = jnp.asarray(inference, jnp.float32).reshape(1, 1)
    out = pl.pallas_call(
        _cluster_body,
        grid=(h // _HT,),
        in_specs=[
            pl.BlockSpec(memory_space=pltpu.SMEM),
            pl.BlockSpec((b, _HT, w, c), lambda i: (0, i, 0, 0)),
            pl.BlockSpec((n, c), lambda i: (0, 0)),
        ],
        out_specs=pl.BlockSpec((n, _HT * w, b), lambda i: (0, i, 0)),
        out_shape=jax.ShapeDtypeStruct((n, h * w, b), jnp.float32),
    )(inf_arr, xt, cluster_probe)
    return out.reshape(n, h, w, b).transpose(3, 0, 1, 2)


# speedup vs baseline: 4.1964x; 1.2639x over previous
import jax
import jax.numpy as jnp
from jax.experimental import pallas as pl
from jax.experimental.pallas import tpu as pltpu

_HT = 2


def _cluster_body(inf_ref, x_ref, w_ref, o_ref, wnt_ref):
    @pl.when(pl.program_id(0) == 0)
    def _prep():
        wv = w_ref[...]
        wn = wv * jax.lax.rsqrt(
            jnp.maximum(jnp.sum(wv * wv, axis=1, keepdims=True), 1e-24))
        wnt_ref[...] = wn.T.astype(jnp.bfloat16)

    b, ht, w, c = x_ref.shape
    xv = x_ref[...].reshape(b * ht * w, c)
    ssq = jnp.sum(xv * xv, axis=1, keepdims=True)
    logits = jnp.dot(xv.astype(jnp.bfloat16), wnt_ref[...],
                     preferred_element_type=jnp.float32)
    ext = jnp.concatenate([logits, ssq], axis=1)
    n1 = ext.shape[1]
    cube = jnp.transpose(ext.reshape(b, ht * w, n1), (2, 1, 0))
    invr = jax.lax.rsqrt(jnp.maximum(cube[n1 - 1:n1], 1e-24))
    o_ref[...] = cube[:n1 - 1] * (invr * inf_ref[0, 0])


def kernel(x, cluster_probe, inference):
    b, c, h, w = x.shape
    n = cluster_probe.shape[0]
    xt = x.transpose(0, 2, 3, 1)
    inf_arr = jnp.asarray(inference, jnp.float32).reshape(1, 1)
    out = pl.pallas_call(
        _cluster_body,
        grid=(h // _HT,),
        in_specs=[
            pl.BlockSpec(memory_space=pltpu.SMEM),
            pl.BlockSpec((b, _HT, w, c), lambda i: (0, i, 0, 0)),
            pl.BlockSpec((n, c), lambda i: (0, 0)),
        ],
        out_specs=pl.BlockSpec((n, _HT * w, b), lambda i: (0, i, 0)),
        out_shape=jax.ShapeDtypeStruct((n, h * w, b), jnp.float32),
        scratch_shapes=[pltpu.VMEM((c, n), jnp.bfloat16)],
        compiler_params=pltpu.CompilerParams(
            vmem_limit_bytes=60 * 1024 * 1024),
    )(inf_arr, xt, cluster_probe)
    return out.reshape(n, h, w, b).transpose(3, 0, 1, 2)
